# Initial kernel scaffold; baseline (speedup 1.0000x reference)
import jax
import jax.numpy as jnp
from jax.experimental import pallas as pl


def kernel(X, edge_index, W_theta, b_theta, a_src, a_dst):
    raise NotImplementedError("write your pallas kernel here")


# speedup vs baseline: 1.4273x; 1.4273x over previous
import functools

import jax
import jax.numpy as jnp
from jax.experimental import pallas as pl


def _elu(x):
    return jnp.where(x > 0, x, jnp.expm1(x))


_BN = 1000


def _h_kernel(x_ref, w_ref, b_ref, asrc_ref, adst_ref, h_ref, ssrc_ref, sdst_ref):
    h = jnp.dot(x_ref[...], w_ref[...], preferred_element_type=jnp.float32)
    h = h + b_ref[...]
    h_ref[...] = h
    ssrc_ref[...] = jnp.sum(h * asrc_ref[...], axis=1, keepdims=True)
    sdst_ref[...] = jnp.sum(h * adst_ref[...], axis=1, keepdims=True)


def _compute_h(X, W_theta, b_theta, a_src, a_dst):
    N, D_in = X.shape
    D_out = W_theta.shape[1]
    grid = (N // _BN,)
    return pl.pallas_call(
        _h_kernel,
        grid=grid,
        in_specs=[
            pl.BlockSpec((_BN, D_in), lambda i: (i, 0)),
            pl.BlockSpec((D_in, D_out), lambda i: (0, 0)),
            pl.BlockSpec((1, D_out), lambda i: (0, 0)),
            pl.BlockSpec((1, D_out), lambda i: (0, 0)),
            pl.BlockSpec((1, D_out), lambda i: (0, 0)),
        ],
        out_specs=[
            pl.BlockSpec((_BN, D_out), lambda i: (i, 0)),
            pl.BlockSpec((_BN, 1), lambda i: (i, 0)),
            pl.BlockSpec((_BN, 1), lambda i: (i, 0)),
        ],
        out_shape=[
            jax.ShapeDtypeStruct((N, D_out), jnp.float32),
            jax.ShapeDtypeStruct((N, 1), jnp.float32),
            jax.ShapeDtypeStruct((N, 1), jnp.float32),
        ],
    )(X, W_theta, b_theta.reshape(1, -1), a_src.reshape(1, -1), a_dst.reshape(1, -1))


def kernel(X, edge_index, W_theta, b_theta, a_src, a_dst):
    H, s_src, s_dst = _compute_h(X, W_theta, b_theta, a_src, a_dst)
    s_src = s_src[:, 0]
    s_dst = s_dst[:, 0]
    e_src = edge_index[0]
    e_dst = edge_index[1]
    N = H.shape[0]
    ex = jnp.exp(_elu(s_src[e_src] + s_dst[e_dst]))
    denom = jax.ops.segment_sum(ex, e_dst, num_segments=N)
    acc = jax.ops.segment_sum(ex[:, None] * H[e_src], e_dst, num_segments=N)
    out = _elu(acc / (denom[:, None] + 1e-16))
    return out


# speedup vs baseline: 3.9082x; 2.7382x over previous
import jax
import jax.numpy as jnp
from jax import lax
from jax.experimental import pallas as pl
from jax.experimental.pallas import tpu as pltpu
from jax.experimental.pallas import tpu_sc as plsc

N = 10000
E = 160000
D = 256
NC = 2
NS = 16
L = 16
HALF = N // NC
DNR = 40
DNS = 48
EPT = E // NS
ECH = 400
NCH = EPT // ECH
B = 80
NB = ECH // B


def _elu(x):
    return jnp.where(x > 0, x, jnp.expm1(x))


_BN = 1000


def _h_body(x_ref, w_ref, b_ref, asrc_ref, adst_ref, h_ref, ssrc_ref, sdst_ref):
    h = jnp.dot(x_ref[...], w_ref[...], preferred_element_type=jnp.float32)
    h = h + b_ref[...]
    h_ref[...] = h
    ssrc_ref[...] = jnp.sum(h * asrc_ref[...], axis=1, keepdims=True)
    sdst_ref[...] = jnp.sum(h * adst_ref[...], axis=1, keepdims=True)


def _compute_h(X, W_theta, b_theta, a_src, a_dst):
    return pl.pallas_call(
        _h_body,
        grid=(N // _BN,),
        in_specs=[
            pl.BlockSpec((_BN, D), lambda i: (i, 0)),
            pl.BlockSpec((D, D), lambda i: (0, 0)),
            pl.BlockSpec((1, D), lambda i: (0, 0)),
            pl.BlockSpec((1, D), lambda i: (0, 0)),
            pl.BlockSpec((1, D), lambda i: (0, 0)),
        ],
        out_specs=[
            pl.BlockSpec((_BN, D), lambda i: (i, 0)),
            pl.BlockSpec((_BN, 1), lambda i: (i, 0)),
            pl.BlockSpec((_BN, 1), lambda i: (i, 0)),
        ],
        out_shape=[
            jax.ShapeDtypeStruct((N, D), jnp.float32),
            jax.ShapeDtypeStruct((N, 1), jnp.float32),
            jax.ShapeDtypeStruct((N, 1), jnp.float32),
        ],
    )(X, W_theta, b_theta.reshape(1, -1), a_src.reshape(1, -1), a_dst.reshape(1, -1))


def _sc_body(ssrc_hbm, sdst_hbm, esrc_hbm, edst_hbm, h_hbm, z_hbm, z2_hbm,
             acclo_hbm, acchi_hbm, dnh_hbm,
             acclo_sp, acchi_sp, dn_sp,
             ssrc_v, sdst_v, esrc_v, edst_v,
             rlo_v, rhi_v, gidx_v, sidx_v, exm_v, dnloc_v, idx48_v, sem):
    c = lax.axis_index("c")
    s = lax.axis_index("s")
    lo = (c * HALF).astype(jnp.int32)
    iota = lax.iota(jnp.int32, L)

    def _zero_dnloc(i, carry):
        for k in range(128 // L):
            dnloc_v[i, pl.ds(k * L, L)] = jnp.zeros((L,), jnp.float32)
        return carry
    lax.fori_loop(0, DNS, _zero_dnloc, None)
    for t in range(DNS // L):
        v = t * L + iota
        idx48_v[pl.ds(t * L, L)] = jnp.where(v < DNR, v, 0)

    r0 = (s * 312).astype(jnp.int32)

    @pl.when(s < 15)
    def _():
        pltpu.sync_copy(z_hbm.at[pl.ds(0, 312)], acclo_sp.at[pl.ds(r0, 312)])
        pltpu.sync_copy(z_hbm.at[pl.ds(0, 312)], acchi_sp.at[pl.ds(r0, 312)])

    @pl.when(s == 15)
    def _():
        pltpu.sync_copy(z_hbm, acclo_sp.at[pl.ds(4680, 320)])
        pltpu.sync_copy(z_hbm, acchi_sp.at[pl.ds(4680, 320)])

    @pl.when(s < 5)
    def _():
        pltpu.sync_copy(z2_hbm.at[pl.ds(8 * s, 8)], dn_sp.at[pl.ds(8 * s, 8)])

    plsc.subcore_barrier()

    pltpu.sync_copy(ssrc_hbm, ssrc_v)
    pltpu.sync_copy(sdst_hbm, sdst_v)
    e0 = s * EPT

    def _batch(b, carry):
        for q in range(B // L):
            base = b * B + q * L
            src16 = esrc_v[pl.ds(base, L)]
            dst16 = edst_v[pl.ds(base, L)]
            vs = plsc.load_gather(ssrc_v, [src16])
            vd = plsc.load_gather(sdst_v, [dst16])
            x = vs + vd
            t = jnp.exp(x)
            ex = jnp.where(x > 0, t, jnp.exp(t - 1.0))
            owned = (dst16 >= lo) & (dst16 < lo + HALF)
            lpos = jnp.where(owned, dst16 - lo, 0)
            plsc.addupdate_scatter(
                dnloc_v,
                [lax.shift_right_logical(lpos, 7), jnp.bitwise_and(lpos, 127)],
                ex, mask=owned)
            gidx_v[pl.ds(q * L, L)] = src16
            sidx_v[pl.ds(q * L, L)] = lpos
            exm_v[pl.ds(q * L, L)] = jnp.where(owned, ex, 0.0)
        pltpu.async_copy(
            h_hbm.at[plsc.Indices(gidx_v), pl.ds(0, 128)], rlo_v, sem).wait()
        pltpu.async_copy(
            h_hbm.at[plsc.Indices(gidx_v), pl.ds(128, 128)], rhi_v, sem).wait()

        def _scale(j, carry2):
            f = plsc.load_gather(exm_v, [jnp.full((L,), j, jnp.int32)])
            for k in range(128 // L):
                rlo_v[j, pl.ds(k * L, L)] = rlo_v[j, pl.ds(k * L, L)] * f
                rhi_v[j, pl.ds(k * L, L)] = rhi_v[j, pl.ds(k * L, L)] * f
            return carry2
        lax.fori_loop(0, B, _scale, None)
        pltpu.sync_copy(rlo_v, acclo_sp.at[plsc.Indices(sidx_v)], add=True)
        pltpu.sync_copy(rhi_v, acchi_sp.at[plsc.Indices(sidx_v)], add=True)
        return carry

    def _chunk(ch, carry):
        pltpu.sync_copy(esrc_hbm.at[pl.ds(e0 + ch * ECH, ECH)], esrc_v)
        pltpu.sync_copy(edst_hbm.at[pl.ds(e0 + ch * ECH, ECH)], edst_v)
        lax.fori_loop(0, NB, _batch, None)
        return carry

    lax.fori_loop(0, NCH, _chunk, None)

    pltpu.sync_copy(dnloc_v, dn_sp.at[plsc.Indices(idx48_v)], add=True)
    plsc.subcore_barrier()

    @pl.when(s < 15)
    def _():
        pltpu.sync_copy(acclo_sp.at[pl.ds(r0, 312)],
                        acclo_hbm.at[pl.ds(c * HALF + r0, 312)])
        pltpu.sync_copy(acchi_sp.at[pl.ds(r0, 312)],
                        acchi_hbm.at[pl.ds(c * HALF + r0, 312)])

    @pl.when(s == 15)
    def _():
        pltpu.sync_copy(acclo_sp.at[pl.ds(4680, 320)],
                        acclo_hbm.at[pl.ds(c * HALF + 4680, 320)])
        pltpu.sync_copy(acchi_sp.at[pl.ds(4680, 320)],
                        acchi_hbm.at[pl.ds(c * HALF + 4680, 320)])

    @pl.when(s < 5)
    def _():
        pltpu.sync_copy(dn_sp.at[pl.ds(8 * s, 8)], dnh_hbm.at[c, pl.ds(8 * s, 8)])


def _sc_aggregate(s_src, s_dst, e_src, e_dst, H):
    mesh = plsc.VectorSubcoreMesh(
        core_axis_name="c", subcore_axis_name="s", num_cores=NC, num_subcores=NS)
    f = pl.kernel(
        _sc_body,
        out_type=[
            jax.ShapeDtypeStruct((N, 128), jnp.float32),
            jax.ShapeDtypeStruct((N, 128), jnp.float32),
            jax.ShapeDtypeStruct((NC, DNR, 128), jnp.float32),
        ],
        mesh=mesh,
        compiler_params=pltpu.CompilerParams(needs_layout_passes=False),
        scratch_types=[
            pltpu.VMEM_SHARED((HALF, 128), jnp.float32),
            pltpu.VMEM_SHARED((HALF, 128), jnp.float32),
            pltpu.VMEM_SHARED((DNR, 128), jnp.float32),
            pltpu.VMEM((N,), jnp.float32),
            pltpu.VMEM((N,), jnp.float32),
            pltpu.VMEM((ECH,), jnp.int32),
            pltpu.VMEM((ECH,), jnp.int32),
            pltpu.VMEM((B, 128), jnp.float32),
            pltpu.VMEM((B, 128), jnp.float32),
            pltpu.VMEM((B,), jnp.int32),
            pltpu.VMEM((B,), jnp.int32),
            pltpu.VMEM((B,), jnp.float32),
            pltpu.VMEM((DNS, 128), jnp.float32),
            pltpu.VMEM((DNS,), jnp.int32),
            pltpu.SemaphoreType.DMA,
        ],
    )
    zeros = jnp.zeros((320, 128), jnp.float32)
    zeros2 = jnp.zeros((DNR, 128), jnp.float32)
    return f(s_src, s_dst, e_src, e_dst, H, zeros, zeros2)


def _final_body(lo_ref, hi_ref, dn_ref, out_ref):
    inv = 1.0 / (dn_ref[...] + 1e-16)
    ylo = lo_ref[...] * inv
    yhi = hi_ref[...] * inv
    out_ref[:, :128] = jnp.where(ylo > 0, ylo, jnp.exp(ylo) - 1.0)
    out_ref[:, 128:] = jnp.where(yhi > 0, yhi, jnp.exp(yhi) - 1.0)


def _final(acclo, acchi, denom):
    return pl.pallas_call(
        _final_body,
        grid=(N // _BN,),
        in_specs=[
            pl.BlockSpec((_BN, 128), lambda i: (i, 0)),
            pl.BlockSpec((_BN, 128), lambda i: (i, 0)),
            pl.BlockSpec((_BN, 1), lambda i: (i, 0)),
        ],
        out_specs=pl.BlockSpec((_BN, D), lambda i: (i, 0)),
        out_shape=jax.ShapeDtypeStruct((N, D), jnp.float32),
    )(acclo, acchi, denom)


def kernel(X, edge_index, W_theta, b_theta, a_src, a_dst):
    H, s_src, s_dst = _compute_h(X, W_theta, b_theta, a_src, a_dst)
    acclo, acchi, dnh = _sc_aggregate(s_src[:, 0], s_dst[:, 0],
                                      edge_index[0], edge_index[1], H)
    denom = dnh.reshape(NC, DNR * 128)[:, :HALF].reshape(N, 1)
    return _final(acclo, acchi, denom)


# speedup vs baseline: 4.5734x; 1.1702x over previous
import jax
import jax.numpy as jnp
from jax import lax
from jax.experimental import pallas as pl
from jax.experimental.pallas import tpu as pltpu
from jax.experimental.pallas import tpu_sc as plsc

N = 10000
E = 160000
D = 256
NC = 2
NS = 16
L = 16
HALF = N // NC
DNR = 80
EPT = E // NS
ECH = 2000
NCH = EPT // ECH
B = 80
NB = EPT // B
BPC = ECH // B
ROWB = B * 128 * 4


def _elu(x):
    return jnp.where(x > 0, x, jnp.expm1(x))


_BN = 1000


def _h_body(x_ref, w_ref, b_ref, asrc_ref, adst_ref, h_ref, ssrc_ref, sdst_ref):
    h = jnp.dot(x_ref[...], w_ref[...], preferred_element_type=jnp.float32)
    h = h + b_ref[...]
    h_ref[...] = h
    ssrc_ref[...] = jnp.sum(h * asrc_ref[...], axis=1, keepdims=True)
    sdst_ref[...] = jnp.sum(h * adst_ref[...], axis=1, keepdims=True)


def _compute_h(X, W_theta, b_theta, a_src, a_dst):
    return pl.pallas_call(
        _h_body,
        grid=(N // _BN,),
        in_specs=[
            pl.BlockSpec((_BN, D), lambda i: (i, 0)),
            pl.BlockSpec((D, D), lambda i: (0, 0)),
            pl.BlockSpec((1, D), lambda i: (0, 0)),
            pl.BlockSpec((1, D), lambda i: (0, 0)),
            pl.BlockSpec((1, D), lambda i: (0, 0)),
        ],
        out_specs=[
            pl.BlockSpec((_BN, D), lambda i: (i, 0)),
            pl.BlockSpec((_BN, 1), lambda i: (i, 0)),
            pl.BlockSpec((_BN, 1), lambda i: (i, 0)),
        ],
        out_shape=[
            jax.ShapeDtypeStruct((N, D), jnp.float32),
            jax.ShapeDtypeStruct((N, 1), jnp.float32),
            jax.ShapeDtypeStruct((N, 1), jnp.float32),
        ],
    )(X, W_theta, b_theta.reshape(1, -1), a_src.reshape(1, -1), a_dst.reshape(1, -1))


def _sc1_body(ssrc_hbm, sdst_hbm, esrc_hbm, edst_hbm, z2_hbm,
              exh_hbm, dnh_hbm,
              dn_sp,
              ssrc_v, sdst_v, esrc_v, edst_v, ex_v, dnloc_v, idx80_v):
    c = lax.axis_index("c")
    s = lax.axis_index("s")
    iota = lax.iota(jnp.int32, L)

    @pl.when(c == 0)
    def _():
        def _zero_dnloc(i, carry):
            for k in range(128 // L):
                dnloc_v[i, pl.ds(k * L, L)] = jnp.zeros((L,), jnp.float32)
            return carry
        lax.fori_loop(0, DNR, _zero_dnloc, None)
        for t in range(DNR // L):
            idx80_v[pl.ds(t * L, L)] = t * L + iota

        @pl.when(s < 10)
        def _():
            pltpu.sync_copy(z2_hbm.at[pl.ds(8 * s, 8)], dn_sp.at[pl.ds(8 * s, 8)])

        pltpu.sync_copy(ssrc_hbm, ssrc_v)
        pltpu.sync_copy(sdst_hbm, sdst_v)
        e0 = s * EPT

        def _group(g, carry):
            base = g * L
            src16 = esrc_v[pl.ds(base, L)]
            dst16 = edst_v[pl.ds(base, L)]
            vs = plsc.load_gather(ssrc_v, [src16])
            vd = plsc.load_gather(sdst_v, [dst16])
            x = vs + vd
            t = jnp.exp(x)
            ex = jnp.where(x > 0, t, jnp.exp(t - 1.0))
            ex_v[pl.ds(base, L)] = ex
            plsc.addupdate_scatter(
                dnloc_v,
                [lax.shift_right_logical(dst16, 7), jnp.bitwise_and(dst16, 127)],
                ex)
            return carry

        def _chunk(ch, carry):
            pltpu.sync_copy(esrc_hbm.at[pl.ds(e0 + ch * ECH, ECH)], esrc_v)
            pltpu.sync_copy(edst_hbm.at[pl.ds(e0 + ch * ECH, ECH)], edst_v)
            lax.fori_loop(0, ECH // L, _group, None)
            pltpu.sync_copy(ex_v, exh_hbm.at[pl.ds(e0 + ch * ECH, ECH)])
            return carry

        lax.fori_loop(0, NCH, _chunk, None)
        plsc.subcore_barrier()
        pltpu.sync_copy(dnloc_v, dn_sp.at[plsc.Indices(idx80_v)], add=True)
        plsc.subcore_barrier()

        @pl.when(s < 10)
        def _():
            pltpu.sync_copy(dn_sp.at[pl.ds(8 * s, 8)], dnh_hbm.at[pl.ds(8 * s, 8)])


def _sc_scores(s_src, s_dst, e_src, e_dst):
    mesh = plsc.VectorSubcoreMesh(
        core_axis_name="c", subcore_axis_name="s", num_cores=NC, num_subcores=NS)
    f = pl.kernel(
        _sc1_body,
        out_type=[
            jax.ShapeDtypeStruct((E,), jnp.float32),
            jax.ShapeDtypeStruct((DNR, 128), jnp.float32),
        ],
        mesh=mesh,
        compiler_params=pltpu.CompilerParams(needs_layout_passes=False),
        scratch_types=[
            pltpu.VMEM_SHARED((DNR, 128), jnp.float32),
            pltpu.VMEM((N,), jnp.float32),
            pltpu.VMEM((N,), jnp.float32),
            pltpu.VMEM((ECH,), jnp.int32),
            pltpu.VMEM((ECH,), jnp.int32),
            pltpu.VMEM((ECH,), jnp.float32),
            pltpu.VMEM((DNR, 128), jnp.float32),
            pltpu.VMEM((DNR,), jnp.int32),
        ],
    )
    zeros2 = jnp.zeros((DNR, 128), jnp.float32)
    return f(s_src, s_dst, e_src, e_dst, zeros2)


def _sc2_body(esrc_hbm, edst_hbm, exh_hbm, h_hbm, z_hbm,
              acclo_hbm, acchi_hbm,
              acclo_sp, acchi_sp,
              esrc_v, edst_v, exch_v,
              rlo0_v, rhi0_v, rlo1_v, rhi1_v,
              gidx0_v, sidx0_v, exm0_v, gidx1_v, sidx1_v, exm1_v,
              semg0, sems0, semg1, sems1):
    c = lax.axis_index("c")
    s = lax.axis_index("s")
    lo = (c * HALF).astype(jnp.int32)

    r0 = (s * 312).astype(jnp.int32)

    @pl.when(s < 15)
    def _():
        pltpu.sync_copy(z_hbm.at[pl.ds(0, 312)], acclo_sp.at[pl.ds(r0, 312)])
        pltpu.sync_copy(z_hbm.at[pl.ds(0, 312)], acchi_sp.at[pl.ds(r0, 312)])

    @pl.when(s == 15)
    def _():
        pltpu.sync_copy(z_hbm, acclo_sp.at[pl.ds(4680, 320)])
        pltpu.sync_copy(z_hbm, acchi_sp.at[pl.ds(4680, 320)])

    plsc.subcore_barrier()

    e0 = s * EPT
    slots = (
        (gidx0_v, sidx0_v, exm0_v, rlo0_v, rhi0_v, semg0, sems0),
        (gidx1_v, sidx1_v, exm1_v, rlo1_v, rhi1_v, semg1, sems1),
    )

    def _wait_scatter(slot):
        gidx_v, sidx_v, exm_v, rlo_v, rhi_v, semg, sems = slot
        pltpu.make_async_copy(
            rlo_v, acclo_sp.at[plsc.Indices(sidx_v)], sems).wait()
        pltpu.make_async_copy(
            rhi_v, acchi_sp.at[plsc.Indices(sidx_v)], sems).wait()

    def _scalar_and_fire(b, slot):
        gidx_v, sidx_v, exm_v, rlo_v, rhi_v, semg, sems = slot

        @pl.when(b % BPC == 0)
        def _():
            ch = b // BPC
            pltpu.sync_copy(esrc_hbm.at[pl.ds(e0 + ch * ECH, ECH)], esrc_v)
            pltpu.sync_copy(edst_hbm.at[pl.ds(e0 + ch * ECH, ECH)], edst_v)
            pltpu.sync_copy(exh_hbm.at[pl.ds(e0 + ch * ECH, ECH)], exch_v)

        off0 = (b % BPC) * B
        for q in range(B // L):
            off = off0 + q * L
            src16 = esrc_v[pl.ds(off, L)]
            dst16 = edst_v[pl.ds(off, L)]
            ex16 = exch_v[pl.ds(off, L)]
            owned = (dst16 >= lo) & (dst16 < lo + HALF)
            lpos = jnp.where(owned, dst16 - lo, jnp.bitwise_and(dst16, 4095))
            gidx_v[pl.ds(q * L, L)] = src16
            sidx_v[pl.ds(q * L, L)] = lpos
            exm_v[pl.ds(q * L, L)] = jnp.where(owned, ex16, 0.0)
        pltpu.async_copy(
            h_hbm.at[plsc.Indices(gidx_v), pl.ds(0, 128)], rlo_v, semg)
        pltpu.async_copy(
            h_hbm.at[plsc.Indices(gidx_v), pl.ds(128, 128)], rhi_v, semg)

    def _wait_gather(slot):
        gidx_v, sidx_v, exm_v, rlo_v, rhi_v, semg, sems = slot
        pltpu.make_async_copy(
            h_hbm.at[plsc.Indices(gidx_v), pl.ds(0, 128)], rlo_v, semg).wait()
        pltpu.make_async_copy(
            h_hbm.at[plsc.Indices(gidx_v), pl.ds(128, 128)], rhi_v, semg).wait()

    def _scale(slot):
        gidx_v, sidx_v, exm_v, rlo_v, rhi_v, semg, sems = slot

        def _row(j, carry2):
            f = plsc.load_gather(exm_v, [jnp.full((L,), j, jnp.int32)])
            for k in range(128 // L):
                rlo_v[j, pl.ds(k * L, L)] = rlo_v[j, pl.ds(k * L, L)] * f
                rhi_v[j, pl.ds(k * L, L)] = rhi_v[j, pl.ds(k * L, L)] * f
            return carry2
        lax.fori_loop(0, B, _row, None)

    def _fire_scatter(slot):
        gidx_v, sidx_v, exm_v, rlo_v, rhi_v, semg, sems = slot
        pltpu.async_copy(
            rlo_v, acclo_sp.at[plsc.Indices(sidx_v)], sems, add=True)
        pltpu.async_copy(
            rhi_v, acchi_sp.at[plsc.Indices(sidx_v)], sems, add=True)

    _scalar_and_fire(jnp.int32(0), slots[0])

    def _step(i, carry):
        b0 = 2 * i
        _wait_gather(slots[0])
        _scale(slots[0])
        _fire_scatter(slots[0])

        @pl.when(b0 >= 1)
        def _():
            _wait_scatter(slots[1])
        _scalar_and_fire(b0 + 1, slots[1])

        _wait_gather(slots[1])
        _scale(slots[1])
        _fire_scatter(slots[1])
        _wait_scatter(slots[0])
        _scalar_and_fire(b0 + 2, slots[0])
        return carry

    lax.fori_loop(0, (NB - 1) // 2, _step, None)
    _wait_gather(slots[0])
    _scale(slots[0])
    _fire_scatter(slots[0])
    _wait_scatter(slots[1])
    _wait_scatter(slots[0])
    plsc.subcore_barrier()

    @pl.when(s < 15)
    def _():
        pltpu.sync_copy(acclo_sp.at[pl.ds(r0, 312)],
                        acclo_hbm.at[pl.ds(c * HALF + r0, 312)])
        pltpu.sync_copy(acchi_sp.at[pl.ds(r0, 312)],
                        acchi_hbm.at[pl.ds(c * HALF + r0, 312)])

    @pl.when(s == 15)
    def _():
        pltpu.sync_copy(acclo_sp.at[pl.ds(4680, 320)],
                        acclo_hbm.at[pl.ds(c * HALF + 4680, 320)])
        pltpu.sync_copy(acchi_sp.at[pl.ds(4680, 320)],
                        acchi_hbm.at[pl.ds(c * HALF + 4680, 320)])


def _sc_aggregate(e_src, e_dst, exh, H):
    mesh = plsc.VectorSubcoreMesh(
        core_axis_name="c", subcore_axis_name="s", num_cores=NC, num_subcores=NS)
    f = pl.kernel(
        _sc2_body,
        out_type=[
            jax.ShapeDtypeStruct((N, 128), jnp.float32),
            jax.ShapeDtypeStruct((N, 128), jnp.float32),
        ],
        mesh=mesh,
        compiler_params=pltpu.CompilerParams(needs_layout_passes=False),
        scratch_types=[
            pltpu.VMEM_SHARED((HALF, 128), jnp.float32),
            pltpu.VMEM_SHARED((HALF, 128), jnp.float32),
            pltpu.VMEM((ECH,), jnp.int32),
            pltpu.VMEM((ECH,), jnp.int32),
            pltpu.VMEM((ECH,), jnp.float32),
            pltpu.VMEM((B, 128), jnp.float32),
            pltpu.VMEM((B, 128), jnp.float32),
            pltpu.VMEM((B, 128), jnp.float32),
            pltpu.VMEM((B, 128), jnp.float32),
            pltpu.VMEM((B,), jnp.int32),
            pltpu.VMEM((B,), jnp.int32),
            pltpu.VMEM((B,), jnp.float32),
            pltpu.VMEM((B,), jnp.int32),
            pltpu.VMEM((B,), jnp.int32),
            pltpu.VMEM((B,), jnp.float32),
            pltpu.SemaphoreType.DMA,
            pltpu.SemaphoreType.DMA,
            pltpu.SemaphoreType.DMA,
            pltpu.SemaphoreType.DMA,
        ],
    )
    zeros = jnp.zeros((320, 128), jnp.float32)
    return f(e_src, e_dst, exh, H, zeros)


def _final_body(lo_ref, hi_ref, dn_ref, out_ref):
    inv = 1.0 / (dn_ref[...] + 1e-16)
    ylo = lo_ref[...] * inv
    yhi = hi_ref[...] * inv
    out_ref[:, :128] = jnp.where(ylo > 0, ylo, jnp.exp(ylo) - 1.0)
    out_ref[:, 128:] = jnp.where(yhi > 0, yhi, jnp.exp(yhi) - 1.0)


def _final(acclo, acchi, denom):
    return pl.pallas_call(
        _final_body,
        grid=(N // _BN,),
        in_specs=[
            pl.BlockSpec((_BN, 128), lambda i: (i, 0)),
            pl.BlockSpec((_BN, 128), lambda i: (i, 0)),
            pl.BlockSpec((_BN, 1), lambda i: (i, 0)),
        ],
        out_specs=pl.BlockSpec((_BN, D), lambda i: (i, 0)),
        out_shape=jax.ShapeDtypeStruct((N, D), jnp.float32),
    )(acclo, acchi, denom)


def kernel(X, edge_index, W_theta, b_theta, a_src, a_dst):
    H, s_src, s_dst = _compute_h(X, W_theta, b_theta, a_src, a_dst)
    e_src = edge_index[0]
    e_dst = edge_index[1]
    exh, dnh = _sc_scores(s_src[:, 0], s_dst[:, 0], e_src, e_dst)
    acclo, acchi = _sc_aggregate(e_src, e_dst, exh, H)
    denom = dnh.reshape(DNR * 128)[:N].reshape(N, 1)
    return _final(acclo, acchi, denom)


# speedup vs baseline: 4.7005x; 1.0278x over previous
import jax
import jax.numpy as jnp
from jax import lax
from jax.experimental import pallas as pl
from jax.experimental.pallas import tpu as pltpu
from jax.experimental.pallas import tpu_sc as plsc

N = 10000
E = 160000
D = 256
NC = 2
NS = 16
L = 16
HALF = N // NC
DNR = 80
EPT = E // NS
ECH = 2000
NCH = EPT // ECH
B = 80
NB = EPT // B
BPC = ECH // B
ROWB = B * 128 * 4


def _elu(x):
    return jnp.where(x > 0, x, jnp.expm1(x))


_BN = 1000


def _h_body(x_ref, w_ref, b_ref, asrc_ref, adst_ref, h_ref, ssrc_ref, sdst_ref):
    h = jnp.dot(x_ref[...], w_ref[...], preferred_element_type=jnp.float32)
    h = h + b_ref[...]
    h_ref[...] = h
    ssrc_ref[...] = jnp.sum(h * asrc_ref[...], axis=1, keepdims=True)
    sdst_ref[...] = jnp.sum(h * adst_ref[...], axis=1, keepdims=True)


def _compute_h(X, W_theta, b_theta, a_src, a_dst):
    return pl.pallas_call(
        _h_body,
        grid=(N // _BN,),
        in_specs=[
            pl.BlockSpec((_BN, D), lambda i: (i, 0)),
            pl.BlockSpec((D, D), lambda i: (0, 0)),
            pl.BlockSpec((1, D), lambda i: (0, 0)),
            pl.BlockSpec((1, D), lambda i: (0, 0)),
            pl.BlockSpec((1, D), lambda i: (0, 0)),
        ],
        out_specs=[
            pl.BlockSpec((_BN, D), lambda i: (i, 0)),
            pl.BlockSpec((_BN, 1), lambda i: (i, 0)),
            pl.BlockSpec((_BN, 1), lambda i: (i, 0)),
        ],
        out_shape=[
            jax.ShapeDtypeStruct((N, D), jnp.float32),
            jax.ShapeDtypeStruct((N, 1), jnp.float32),
            jax.ShapeDtypeStruct((N, 1), jnp.float32),
        ],
    )(X, W_theta, b_theta.reshape(1, -1), a_src.reshape(1, -1), a_dst.reshape(1, -1))


def _sc1_body(ssrc_hbm, sdst_hbm, esrc_hbm, edst_hbm, z2_hbm,
              exh_hbm, dnh_hbm,
              dn_sp,
              ssrc_v, sdst_v, esrc_v, edst_v, ex_v, dnloc_v, idx80_v):
    c = lax.axis_index("c")
    s = lax.axis_index("s")
    iota = lax.iota(jnp.int32, L)

    @pl.when(c == 0)
    def _():
        def _zero_dnloc(i, carry):
            for k in range(128 // L):
                dnloc_v[i, pl.ds(k * L, L)] = jnp.zeros((L,), jnp.float32)
            return carry
        lax.fori_loop(0, DNR, _zero_dnloc, None)
        for t in range(DNR // L):
            idx80_v[pl.ds(t * L, L)] = t * L + iota

        @pl.when(s < 10)
        def _():
            pltpu.sync_copy(z2_hbm.at[pl.ds(8 * s, 8)], dn_sp.at[pl.ds(8 * s, 8)])

        pltpu.sync_copy(ssrc_hbm, ssrc_v)
        pltpu.sync_copy(sdst_hbm, sdst_v)
        e0 = s * EPT

        def _group(g, carry):
            base = g * L
            src16 = esrc_v[pl.ds(base, L)]
            dst16 = edst_v[pl.ds(base, L)]
            vs = plsc.load_gather(ssrc_v, [src16])
            vd = plsc.load_gather(sdst_v, [dst16])
            x = vs + vd
            t = jnp.exp(x)
            ex = jnp.where(x > 0, t, jnp.exp(t - 1.0))
            ex_v[pl.ds(base, L)] = ex
            plsc.addupdate_scatter(
                dnloc_v,
                [lax.shift_right_logical(dst16, 7), jnp.bitwise_and(dst16, 127)],
                ex)
            return carry

        def _chunk(ch, carry):
            pltpu.sync_copy(esrc_hbm.at[pl.ds(e0 + ch * ECH, ECH)], esrc_v)
            pltpu.sync_copy(edst_hbm.at[pl.ds(e0 + ch * ECH, ECH)], edst_v)
            lax.fori_loop(0, ECH // L, _group, None)
            pltpu.sync_copy(ex_v, exh_hbm.at[pl.ds(e0 + ch * ECH, ECH)])
            return carry

        lax.fori_loop(0, NCH, _chunk, None)
        plsc.subcore_barrier()
        pltpu.sync_copy(dnloc_v, dn_sp.at[plsc.Indices(idx80_v)], add=True)
        plsc.subcore_barrier()

        @pl.when(s < 10)
        def _():
            pltpu.sync_copy(dn_sp.at[pl.ds(8 * s, 8)], dnh_hbm.at[pl.ds(8 * s, 8)])


def _sc_scores(s_src, s_dst, e_src, e_dst):
    mesh = plsc.VectorSubcoreMesh(
        core_axis_name="c", subcore_axis_name="s", num_cores=NC, num_subcores=NS)
    f = pl.kernel(
        _sc1_body,
        out_type=[
            jax.ShapeDtypeStruct((E,), jnp.float32),
            jax.ShapeDtypeStruct((DNR, 128), jnp.float32),
        ],
        mesh=mesh,
        compiler_params=pltpu.CompilerParams(needs_layout_passes=False),
        scratch_types=[
            pltpu.VMEM_SHARED((DNR, 128), jnp.float32),
            pltpu.VMEM((N,), jnp.float32),
            pltpu.VMEM((N,), jnp.float32),
            pltpu.VMEM((ECH,), jnp.int32),
            pltpu.VMEM((ECH,), jnp.int32),
            pltpu.VMEM((ECH,), jnp.float32),
            pltpu.VMEM((DNR, 128), jnp.float32),
            pltpu.VMEM((DNR,), jnp.int32),
        ],
    )
    zeros2 = jnp.zeros((DNR, 128), jnp.float32)
    return f(s_src, s_dst, e_src, e_dst, zeros2)


def _sc2_body(esrc_hbm, edst_hbm, exh_hbm, h_hbm, z_hbm,
              acclo_hbm, acchi_hbm,
              acclo_sp, acchi_sp,
              esrc_v, edst_v, exch_v,
              rlo0_v, rhi0_v, rlo1_v, rhi1_v,
              gidx0_v, sidx0_v, exm0_v, gidx1_v, sidx1_v, exm1_v,
              semg0, sems0, semg1, sems1):
    c = lax.axis_index("c")
    s = lax.axis_index("s")
    lo = (c * HALF).astype(jnp.int32)

    r0 = (s * 312).astype(jnp.int32)

    @pl.when(s < 15)
    def _():
        pltpu.sync_copy(z_hbm.at[pl.ds(0, 312)], acclo_sp.at[pl.ds(r0, 312)])
        pltpu.sync_copy(z_hbm.at[pl.ds(0, 312)], acchi_sp.at[pl.ds(r0, 312)])

    @pl.when(s == 15)
    def _():
        pltpu.sync_copy(z_hbm, acclo_sp.at[pl.ds(4680, 320)])
        pltpu.sync_copy(z_hbm, acchi_sp.at[pl.ds(4680, 320)])

    plsc.subcore_barrier()

    e0 = s * EPT
    slots = (
        (gidx0_v, sidx0_v, exm0_v, rlo0_v, rhi0_v, semg0, sems0),
        (gidx1_v, sidx1_v, exm1_v, rlo1_v, rhi1_v, semg1, sems1),
    )

    def _wait_scatter(slot):
        gidx_v, sidx_v, exm_v, rlo_v, rhi_v, semg, sems = slot
        pltpu.make_async_copy(
            rlo_v, acclo_sp.at[plsc.Indices(sidx_v)], sems).wait()
        pltpu.make_async_copy(
            rhi_v, acchi_sp.at[plsc.Indices(sidx_v)], sems).wait()

    def _scalar_and_fire(b, slot):
        gidx_v, sidx_v, exm_v, rlo_v, rhi_v, semg, sems = slot

        @pl.when(b % BPC == 0)
        def _():
            ch = b // BPC
            pltpu.sync_copy(esrc_hbm.at[pl.ds(e0 + ch * ECH, ECH)], esrc_v)
            pltpu.sync_copy(edst_hbm.at[pl.ds(e0 + ch * ECH, ECH)], edst_v)
            pltpu.sync_copy(exh_hbm.at[pl.ds(e0 + ch * ECH, ECH)], exch_v)

        off0 = (b % BPC) * B
        for q in range(B // L):
            off = off0 + q * L
            src16 = esrc_v[pl.ds(off, L)]
            dst16 = edst_v[pl.ds(off, L)]
            ex16 = exch_v[pl.ds(off, L)]
            owned = (dst16 >= lo) & (dst16 < lo + HALF)
            lpos = jnp.where(owned, dst16 - lo, jnp.bitwise_and(dst16, 4095))
            gidx_v[pl.ds(q * L, L)] = src16
            sidx_v[pl.ds(q * L, L)] = lpos
            exm_v[pl.ds(q * L, L)] = jnp.where(owned, ex16, 0.0)
        pltpu.async_copy(
            h_hbm.at[plsc.Indices(gidx_v), pl.ds(0, 128)], rlo_v, semg)
        pltpu.async_copy(
            h_hbm.at[plsc.Indices(gidx_v), pl.ds(128, 128)], rhi_v, semg)

    def _wait_gather(slot):
        gidx_v, sidx_v, exm_v, rlo_v, rhi_v, semg, sems = slot
        pltpu.make_async_copy(
            h_hbm.at[plsc.Indices(gidx_v), pl.ds(0, 128)], rlo_v, semg).wait()
        pltpu.make_async_copy(
            h_hbm.at[plsc.Indices(gidx_v), pl.ds(128, 128)], rhi_v, semg).wait()

    def _scale(slot):
        gidx_v, sidx_v, exm_v, rlo_v, rhi_v, semg, sems = slot

        def _row(j, carry2):
            f = plsc.load_gather(exm_v, [jnp.full((L,), j, jnp.int32)])
            for k in range(128 // L):
                rlo_v[j, pl.ds(k * L, L)] = rlo_v[j, pl.ds(k * L, L)] * f
                rhi_v[j, pl.ds(k * L, L)] = rhi_v[j, pl.ds(k * L, L)] * f
            return carry2
        lax.fori_loop(0, B, _row, None)

    def _fire_scatter(slot):
        gidx_v, sidx_v, exm_v, rlo_v, rhi_v, semg, sems = slot
        pltpu.async_copy(
            rlo_v, acclo_sp.at[plsc.Indices(sidx_v)], sems, add=True)
        pltpu.async_copy(
            rhi_v, acchi_sp.at[plsc.Indices(sidx_v)], sems, add=True)

    _scalar_and_fire(jnp.int32(0), slots[0])
    _scalar_and_fire(jnp.int32(1), slots[1])

    def _step(i, carry):
        b0 = 2 * i
        _wait_gather(slots[0])
        _scale(slots[0])
        _fire_scatter(slots[0])

        @pl.when(b0 + 2 < NB)
        def _():
            _wait_scatter(slots[0])
            _scalar_and_fire(b0 + 2, slots[0])

        _wait_gather(slots[1])
        _scale(slots[1])
        _fire_scatter(slots[1])

        @pl.when(b0 + 3 < NB)
        def _():
            _wait_scatter(slots[1])
            _scalar_and_fire(b0 + 3, slots[1])
        return carry

    lax.fori_loop(0, NB // 2, _step, None)
    _wait_gather(slots[0])
    _scale(slots[0])
    _fire_scatter(slots[0])
    _wait_scatter(slots[1])
    _wait_scatter(slots[0])
    plsc.subcore_barrier()

    @pl.when(s < 15)
    def _():
        pltpu.sync_copy(acclo_sp.at[pl.ds(r0, 312)],
                        acclo_hbm.at[pl.ds(c * HALF + r0, 312)])
        pltpu.sync_copy(acchi_sp.at[pl.ds(r0, 312)],
                        acchi_hbm.at[pl.ds(c * HALF + r0, 312)])

    @pl.when(s == 15)
    def _():
        pltpu.sync_copy(acclo_sp.at[pl.ds(4680, 320)],
                        acclo_hbm.at[pl.ds(c * HALF + 4680, 320)])
        pltpu.sync_copy(acchi_sp.at[pl.ds(4680, 320)],
                        acchi_hbm.at[pl.ds(c * HALF + 4680, 320)])


def _sc_aggregate(e_src, e_dst, exh, H):
    mesh = plsc.VectorSubcoreMesh(
        core_axis_name="c", subcore_axis_name="s", num_cores=NC, num_subcores=NS)
    f = pl.kernel(
        _sc2_body,
        out_type=[
            jax.ShapeDtypeStruct((N, 128), jnp.float32),
            jax.ShapeDtypeStruct((N, 128), jnp.float32),
        ],
        mesh=mesh,
        compiler_params=pltpu.CompilerParams(needs_layout_passes=False),
        scratch_types=[
            pltpu.VMEM_SHARED((HALF, 128), jnp.float32),
            pltpu.VMEM_SHARED((HALF, 128), jnp.float32),
            pltpu.VMEM((ECH,), jnp.int32),
            pltpu.VMEM((ECH,), jnp.int32),
            pltpu.VMEM((ECH,), jnp.float32),
            pltpu.VMEM((B, 128), jnp.float32),
            pltpu.VMEM((B, 128), jnp.float32),
            pltpu.VMEM((B, 128), jnp.float32),
            pltpu.VMEM((B, 128), jnp.float32),
            pltpu.VMEM((B,), jnp.int32),
            pltpu.VMEM((B,), jnp.int32),
            pltpu.VMEM((B,), jnp.float32),
            pltpu.VMEM((B,), jnp.int32),
            pltpu.VMEM((B,), jnp.int32),
            pltpu.VMEM((B,), jnp.float32),
            pltpu.SemaphoreType.DMA,
            pltpu.SemaphoreType.DMA,
            pltpu.SemaphoreType.DMA,
            pltpu.SemaphoreType.DMA,
        ],
    )
    zeros = jnp.zeros((320, 128), jnp.float32)
    return f(e_src, e_dst, exh, H, zeros)


def _final_body(lo_ref, hi_ref, dn_ref, out_ref):
    inv = 1.0 / (dn_ref[...] + 1e-16)
    ylo = lo_ref[...] * inv
    yhi = hi_ref[...] * inv
    out_ref[:, :128] = jnp.where(ylo > 0, ylo, jnp.exp(ylo) - 1.0)
    out_ref[:, 128:] = jnp.where(yhi > 0, yhi, jnp.exp(yhi) - 1.0)


def _final(acclo, acchi, denom):
    return pl.pallas_call(
        _final_body,
        grid=(N // _BN,),
        in_specs=[
            pl.BlockSpec((_BN, 128), lambda i: (i, 0)),
            pl.BlockSpec((_BN, 128), lambda i: (i, 0)),
            pl.BlockSpec((_BN, 1), lambda i: (i, 0)),
        ],
        out_specs=pl.BlockSpec((_BN, D), lambda i: (i, 0)),
        out_shape=jax.ShapeDtypeStruct((N, D), jnp.float32),
    )(acclo, acchi, denom)


def kernel(X, edge_index, W_theta, b_theta, a_src, a_dst):
    H, s_src, s_dst = _compute_h(X, W_theta, b_theta, a_src, a_dst)
    e_src = edge_index[0]
    e_dst = edge_index[1]
    exh, dnh = _sc_scores(s_src[:, 0], s_dst[:, 0], e_src, e_dst)
    acclo, acchi = _sc_aggregate(e_src, e_dst, exh, H)
    denom = dnh.reshape(DNR * 128)[:N].reshape(N, 1)
    return _final(acclo, acchi, denom)


# speedup vs baseline: 10.5199x; 2.2381x over previous
import jax
import jax.numpy as jnp
from jax import lax
from jax.experimental import pallas as pl
from jax.experimental.pallas import tpu as pltpu
from jax.experimental.pallas import tpu_sc as plsc

N = 10000
E = 160000
D = 256
NC = 2
NS = 16
L = 16
HALF = N // NC
DNR = 80
EPT = E // NS
ECH = 2000
NCH = EPT // ECH
B = 80
NB = EPT // B
BPC = ECH // B
ROWB = B * 128 * 4


def _elu(x):
    return jnp.where(x > 0, x, jnp.expm1(x))


_BN = 1000


def _h_body(x_ref, w_ref, b_ref, asrc_ref, adst_ref, h_ref, ssrc_ref, sdst_ref):
    h = jnp.dot(x_ref[...], w_ref[...], preferred_element_type=jnp.float32)
    h = h + b_ref[...]
    h_ref[...] = h
    ssrc_ref[...] = jnp.sum(h * asrc_ref[...], axis=1, keepdims=True)
    sdst_ref[...] = jnp.sum(h * adst_ref[...], axis=1, keepdims=True)


def _compute_h(X, W_theta, b_theta, a_src, a_dst):
    return pl.pallas_call(
        _h_body,
        grid=(N // _BN,),
        in_specs=[
            pl.BlockSpec((_BN, D), lambda i: (i, 0)),
            pl.BlockSpec((D, D), lambda i: (0, 0)),
            pl.BlockSpec((1, D), lambda i: (0, 0)),
            pl.BlockSpec((1, D), lambda i: (0, 0)),
            pl.BlockSpec((1, D), lambda i: (0, 0)),
        ],
        out_specs=[
            pl.BlockSpec((_BN, D), lambda i: (i, 0)),
            pl.BlockSpec((_BN, 1), lambda i: (i, 0)),
            pl.BlockSpec((_BN, 1), lambda i: (i, 0)),
        ],
        out_shape=[
            jax.ShapeDtypeStruct((N, D), jnp.float32),
            jax.ShapeDtypeStruct((N, 1), jnp.float32),
            jax.ShapeDtypeStruct((N, 1), jnp.float32),
        ],
    )(X, W_theta, b_theta.reshape(1, -1), a_src.reshape(1, -1), a_dst.reshape(1, -1))


def _sc1_body(ssrc_hbm, sdst_hbm, esrc_hbm, edst_hbm, z2_hbm,
              exh_hbm, dnh_hbm,
              dn_sp,
              ssrc_v, sdst_v, esrc_v, edst_v, ex_v, dnloc_v, idx80_v):
    c = lax.axis_index("c")
    s = lax.axis_index("s")
    iota = lax.iota(jnp.int32, L)

    @pl.when(c == 0)
    def _():
        def _zero_dnloc(i, carry):
            for k in range(128 // L):
                dnloc_v[i, pl.ds(k * L, L)] = jnp.zeros((L,), jnp.float32)
            return carry
        lax.fori_loop(0, DNR, _zero_dnloc, None)
        for t in range(DNR // L):
            idx80_v[pl.ds(t * L, L)] = t * L + iota

        @pl.when(s < 10)
        def _():
            pltpu.sync_copy(z2_hbm.at[pl.ds(8 * s, 8)], dn_sp.at[pl.ds(8 * s, 8)])

        pltpu.sync_copy(ssrc_hbm, ssrc_v)
        pltpu.sync_copy(sdst_hbm, sdst_v)
        e0 = s * EPT

        def _group(g, carry):
            base = g * L
            src16 = esrc_v[pl.ds(base, L)]
            dst16 = edst_v[pl.ds(base, L)]
            vs = plsc.load_gather(ssrc_v, [src16])
            vd = plsc.load_gather(sdst_v, [dst16])
            x = vs + vd
            t = jnp.exp(x)
            ex = jnp.where(x > 0, t, jnp.exp(t - 1.0))
            ex_v[pl.ds(base, L)] = ex
            plsc.addupdate_scatter(
                dnloc_v,
                [lax.shift_right_logical(dst16, 7), jnp.bitwise_and(dst16, 127)],
                ex)
            return carry

        def _chunk(ch, carry):
            pltpu.sync_copy(esrc_hbm.at[pl.ds(e0 + ch * ECH, ECH)], esrc_v)
            pltpu.sync_copy(edst_hbm.at[pl.ds(e0 + ch * ECH, ECH)], edst_v)
            lax.fori_loop(0, ECH // L, _group, None)
            pltpu.sync_copy(ex_v, exh_hbm.at[pl.ds(e0 + ch * ECH, ECH)])
            return carry

        lax.fori_loop(0, NCH, _chunk, None)
        plsc.subcore_barrier()
        pltpu.sync_copy(dnloc_v, dn_sp.at[plsc.Indices(idx80_v)], add=True)
        plsc.subcore_barrier()

        @pl.when(s < 10)
        def _():
            pltpu.sync_copy(dn_sp.at[pl.ds(8 * s, 8)], dnh_hbm.at[pl.ds(8 * s, 8)])


def _sc_scores(s_src, s_dst, e_src, e_dst):
    mesh = plsc.VectorSubcoreMesh(
        core_axis_name="c", subcore_axis_name="s", num_cores=NC, num_subcores=NS)
    f = pl.kernel(
        _sc1_body,
        out_type=[
            jax.ShapeDtypeStruct((E,), jnp.float32),
            jax.ShapeDtypeStruct((DNR, 128), jnp.float32),
        ],
        mesh=mesh,
        compiler_params=pltpu.CompilerParams(needs_layout_passes=False),
        scratch_types=[
            pltpu.VMEM_SHARED((DNR, 128), jnp.float32),
            pltpu.VMEM((N,), jnp.float32),
            pltpu.VMEM((N,), jnp.float32),
            pltpu.VMEM((ECH,), jnp.int32),
            pltpu.VMEM((ECH,), jnp.int32),
            pltpu.VMEM((ECH,), jnp.float32),
            pltpu.VMEM((DNR, 128), jnp.float32),
            pltpu.VMEM((DNR,), jnp.int32),
        ],
    )
    zeros2 = jnp.zeros((DNR, 128), jnp.float32)
    return f(s_src, s_dst, e_src, e_dst, zeros2)


def _sc2_body(esrc_hbm, edst_hbm, exh_hbm, h_hbm, z_hbm,
              acclo_hbm, acchi_hbm,
              acclo_sp, acchi_sp,
              esrc_v, edst_v, exch_v,
              rlo0_v, rhi0_v, rlo1_v, rhi1_v,
              gidx0_v, sidx0_v, exm0_v, gidx1_v, sidx1_v, exm1_v,
              semg0, sems0, semg1, sems1):
    c = lax.axis_index("c")
    s = lax.axis_index("s")
    lo = (c * HALF).astype(jnp.int32)

    r0 = (s * 312).astype(jnp.int32)

    @pl.when(s < 15)
    def _():
        pltpu.sync_copy(z_hbm.at[pl.ds(0, 312)], acclo_sp.at[pl.ds(r0, 312)])
        pltpu.sync_copy(z_hbm.at[pl.ds(0, 312)], acchi_sp.at[pl.ds(r0, 312)])

    @pl.when(s == 15)
    def _():
        pltpu.sync_copy(z_hbm, acclo_sp.at[pl.ds(4680, 320)])
        pltpu.sync_copy(z_hbm, acchi_sp.at[pl.ds(4680, 320)])

    plsc.subcore_barrier()

    e0 = s * EPT
    slots = (
        (gidx0_v, sidx0_v, exm0_v, rlo0_v, rhi0_v, semg0, sems0),
        (gidx1_v, sidx1_v, exm1_v, rlo1_v, rhi1_v, semg1, sems1),
    )

    def _wait_scatter(slot):
        gidx_v, sidx_v, exm_v, rlo_v, rhi_v, semg, sems = slot
        pltpu.make_async_copy(
            rlo_v, acclo_sp.at[plsc.Indices(sidx_v)], sems).wait()
        pltpu.make_async_copy(
            rhi_v, acchi_sp.at[plsc.Indices(sidx_v)], sems).wait()

    def _scalar_and_fire(b, slot):
        gidx_v, sidx_v, exm_v, rlo_v, rhi_v, semg, sems = slot

        @pl.when(b % BPC == 0)
        def _():
            ch = b // BPC
            pltpu.sync_copy(esrc_hbm.at[pl.ds(e0 + ch * ECH, ECH)], esrc_v)
            pltpu.sync_copy(edst_hbm.at[pl.ds(e0 + ch * ECH, ECH)], edst_v)
            pltpu.sync_copy(exh_hbm.at[pl.ds(e0 + ch * ECH, ECH)], exch_v)

        off0 = (b % BPC) * B
        for q in range(B // L):
            off = off0 + q * L
            src16 = esrc_v[pl.ds(off, L)]
            dst16 = edst_v[pl.ds(off, L)]
            ex16 = exch_v[pl.ds(off, L)]
            owned = (dst16 >= lo) & (dst16 < lo + HALF)
            lpos = jnp.where(owned, dst16 - lo, jnp.bitwise_and(dst16, 4095))
            gidx_v[pl.ds(q * L, L)] = src16
            sidx_v[pl.ds(q * L, L)] = lpos
            exm_v[pl.ds(q * L, L)] = jnp.where(owned, ex16, 0.0)
        pltpu.async_copy(
            h_hbm.at[plsc.Indices(gidx_v), pl.ds(0, 128)], rlo_v, semg)
        pltpu.async_copy(
            h_hbm.at[plsc.Indices(gidx_v), pl.ds(128, 128)], rhi_v, semg)

    def _wait_gather(slot):
        gidx_v, sidx_v, exm_v, rlo_v, rhi_v, semg, sems = slot
        pltpu.make_async_copy(
            h_hbm.at[plsc.Indices(gidx_v), pl.ds(0, 128)], rlo_v, semg).wait()
        pltpu.make_async_copy(
            h_hbm.at[plsc.Indices(gidx_v), pl.ds(128, 128)], rhi_v, semg).wait()

    def _scale(slot):
        gidx_v, sidx_v, exm_v, rlo_v, rhi_v, semg, sems = slot

        def _rowgrp(g, carry2):
            for jo in range(L):
                j = g * L + jo
                f = plsc.load_gather(exm_v, [jnp.full((L,), jo, jnp.int32) + g * L])
                for k in range(128 // L):
                    rlo_v[j, pl.ds(k * L, L)] = rlo_v[j, pl.ds(k * L, L)] * f
                    rhi_v[j, pl.ds(k * L, L)] = rhi_v[j, pl.ds(k * L, L)] * f
            return carry2
        lax.fori_loop(0, B // L, _rowgrp, None)

    def _fire_scatter(slot):
        gidx_v, sidx_v, exm_v, rlo_v, rhi_v, semg, sems = slot
        pltpu.async_copy(
            rlo_v, acclo_sp.at[plsc.Indices(sidx_v)], sems, add=True)
        pltpu.async_copy(
            rhi_v, acchi_sp.at[plsc.Indices(sidx_v)], sems, add=True)

    _scalar_and_fire(jnp.int32(0), slots[0])
    _scalar_and_fire(jnp.int32(1), slots[1])

    def _step(i, carry):
        b0 = 2 * i
        _wait_gather(slots[0])
        _scale(slots[0])
        _fire_scatter(slots[0])

        @pl.when(b0 + 2 < NB)
        def _():
            _wait_scatter(slots[0])
            _scalar_and_fire(b0 + 2, slots[0])

        _wait_gather(slots[1])
        _scale(slots[1])
        _fire_scatter(slots[1])

        @pl.when(b0 + 3 < NB)
        def _():
            _wait_scatter(slots[1])
            _scalar_and_fire(b0 + 3, slots[1])
        return carry

    lax.fori_loop(0, NB // 2, _step, None)
    _wait_gather(slots[0])
    _scale(slots[0])
    _fire_scatter(slots[0])
    _wait_scatter(slots[1])
    _wait_scatter(slots[0])
    plsc.subcore_barrier()

    @pl.when(s < 15)
    def _():
        pltpu.sync_copy(acclo_sp.at[pl.ds(r0, 312)],
                        acclo_hbm.at[pl.ds(c * HALF + r0, 312)])
        pltpu.sync_copy(acchi_sp.at[pl.ds(r0, 312)],
                        acchi_hbm.at[pl.ds(c * HALF + r0, 312)])

    @pl.when(s == 15)
    def _():
        pltpu.sync_copy(acclo_sp.at[pl.ds(4680, 320)],
                        acclo_hbm.at[pl.ds(c * HALF + 4680, 320)])
        pltpu.sync_copy(acchi_sp.at[pl.ds(4680, 320)],
                        acchi_hbm.at[pl.ds(c * HALF + 4680, 320)])


def _sc_aggregate(e_src, e_dst, exh, H):
    mesh = plsc.VectorSubcoreMesh(
        core_axis_name="c", subcore_axis_name="s", num_cores=NC, num_subcores=NS)
    f = pl.kernel(
        _sc2_body,
        out_type=[
            jax.ShapeDtypeStruct((N, 128), jnp.float32),
            jax.ShapeDtypeStruct((N, 128), jnp.float32),
        ],
        mesh=mesh,
        compiler_params=pltpu.CompilerParams(needs_layout_passes=False),
        scratch_types=[
            pltpu.VMEM_SHARED((HALF, 128), jnp.float32),
            pltpu.VMEM_SHARED((HALF, 128), jnp.float32),
            pltpu.VMEM((ECH,), jnp.int32),
            pltpu.VMEM((ECH,), jnp.int32),
            pltpu.VMEM((ECH,), jnp.float32),
            pltpu.VMEM((B, 128), jnp.float32),
            pltpu.VMEM((B, 128), jnp.float32),
            pltpu.VMEM((B, 128), jnp.float32),
            pltpu.VMEM((B, 128), jnp.float32),
            pltpu.VMEM((B,), jnp.int32),
            pltpu.VMEM((B,), jnp.int32),
            pltpu.VMEM((B,), jnp.float32),
            pltpu.VMEM((B,), jnp.int32),
            pltpu.VMEM((B,), jnp.int32),
            pltpu.VMEM((B,), jnp.float32),
            pltpu.SemaphoreType.DMA,
            pltpu.SemaphoreType.DMA,
            pltpu.SemaphoreType.DMA,
            pltpu.SemaphoreType.DMA,
        ],
    )
    zeros = jnp.zeros((320, 128), jnp.float32)
    return f(e_src, e_dst, exh, H, zeros)


def _final_body(lo_ref, hi_ref, dn_ref, out_ref):
    inv = 1.0 / (dn_ref[...] + 1e-16)
    ylo = lo_ref[...] * inv
    yhi = hi_ref[...] * inv
    out_ref[:, :128] = jnp.where(ylo > 0, ylo, jnp.exp(ylo) - 1.0)
    out_ref[:, 128:] = jnp.where(yhi > 0, yhi, jnp.exp(yhi) - 1.0)


def _final(acclo, acchi, denom):
    return pl.pallas_call(
        _final_body,
        grid=(N // _BN,),
        in_specs=[
            pl.BlockSpec((_BN, 128), lambda i: (i, 0)),
            pl.BlockSpec((_BN, 128), lambda i: (i, 0)),
            pl.BlockSpec((_BN, 1), lambda i: (i, 0)),
        ],
        out_specs=pl.BlockSpec((_BN, D), lambda i: (i, 0)),
        out_shape=jax.ShapeDtypeStruct((N, D), jnp.float32),
    )(acclo, acchi, denom)


def kernel(X, edge_index, W_theta, b_theta, a_src, a_dst):
    H, s_src, s_dst = _compute_h(X, W_theta, b_theta, a_src, a_dst)
    e_src = edge_index[0]
    e_dst = edge_index[1]
    exh, dnh = _sc_scores(s_src[:, 0], s_dst[:, 0], e_src, e_dst)
    acclo, acchi = _sc_aggregate(e_src, e_dst, exh, H)
    denom = dnh.reshape(DNR * 128)[:N].reshape(N, 1)
    return _final(acclo, acchi, denom)


# speedup vs baseline: 11.3991x; 1.0836x over previous
import jax
import jax.numpy as jnp
from jax import lax
from jax.experimental import pallas as pl
from jax.experimental.pallas import tpu as pltpu
from jax.experimental.pallas import tpu_sc as plsc

N = 10000
E = 160000
D = 256
NC = 2
NS = 16
L = 16
HALF = N // NC
DNR = 80
EPT = E // NS
ECH = 2000
NCH = EPT // ECH
B = 80
NB = EPT // B
BPC = ECH // B
ROWB = B * 128 * 4


def _elu(x):
    return jnp.where(x > 0, x, jnp.expm1(x))


_BN = 1000


def _h_body(x_ref, w_ref, b_ref, asrc_ref, adst_ref, h_ref, ssrc_ref, sdst_ref):
    h = jnp.dot(x_ref[...], w_ref[...], preferred_element_type=jnp.float32)
    h = h + b_ref[...]
    h_ref[...] = h
    ssrc_ref[...] = jnp.sum(h * asrc_ref[...], axis=1, keepdims=True)
    sdst_ref[...] = jnp.sum(h * adst_ref[...], axis=1, keepdims=True)


def _compute_h(X, W_theta, b_theta, a_src, a_dst):
    return pl.pallas_call(
        _h_body,
        grid=(N // _BN,),
        in_specs=[
            pl.BlockSpec((_BN, D), lambda i: (i, 0)),
            pl.BlockSpec((D, D), lambda i: (0, 0)),
            pl.BlockSpec((1, D), lambda i: (0, 0)),
            pl.BlockSpec((1, D), lambda i: (0, 0)),
            pl.BlockSpec((1, D), lambda i: (0, 0)),
        ],
        out_specs=[
            pl.BlockSpec((_BN, D), lambda i: (i, 0)),
            pl.BlockSpec((_BN, 1), lambda i: (i, 0)),
            pl.BlockSpec((_BN, 1), lambda i: (i, 0)),
        ],
        out_shape=[
            jax.ShapeDtypeStruct((N, D), jnp.float32),
            jax.ShapeDtypeStruct((N, 1), jnp.float32),
            jax.ShapeDtypeStruct((N, 1), jnp.float32),
        ],
    )(X, W_theta, b_theta.reshape(1, -1), a_src.reshape(1, -1), a_dst.reshape(1, -1))


def _sc1_body(ssrc_hbm, sdst_hbm, esrc_hbm, edst_hbm, z2_hbm,
              exh_hbm, dnh_hbm,
              dn_sp,
              ssrc_v, sdst_v, esrc_v, edst_v, ex_v, dnloc_v, idx80_v):
    c = lax.axis_index("c")
    s = lax.axis_index("s")
    iota = lax.iota(jnp.int32, L)

    @pl.when(c == 0)
    def _():
        def _zero_dnloc(i, carry):
            for k in range(128 // L):
                dnloc_v[i, pl.ds(k * L, L)] = jnp.zeros((L,), jnp.float32)
            return carry
        lax.fori_loop(0, DNR, _zero_dnloc, None)
        for t in range(DNR // L):
            idx80_v[pl.ds(t * L, L)] = t * L + iota

        @pl.when(s < 10)
        def _():
            pltpu.sync_copy(z2_hbm.at[pl.ds(8 * s, 8)], dn_sp.at[pl.ds(8 * s, 8)])

        pltpu.sync_copy(ssrc_hbm, ssrc_v)
        pltpu.sync_copy(sdst_hbm, sdst_v)
        e0 = s * EPT

        def _group(g, carry):
            base = g * L
            src16 = esrc_v[pl.ds(base, L)]
            dst16 = edst_v[pl.ds(base, L)]
            vs = plsc.load_gather(ssrc_v, [src16])
            vd = plsc.load_gather(sdst_v, [dst16])
            x = vs + vd
            t = jnp.exp(x)
            ex = jnp.where(x > 0, t, jnp.exp(t - 1.0))
            ex_v[pl.ds(base, L)] = ex
            plsc.addupdate_scatter(
                dnloc_v,
                [lax.shift_right_logical(dst16, 7), jnp.bitwise_and(dst16, 127)],
                ex)
            return carry

        def _chunk(ch, carry):
            pltpu.sync_copy(esrc_hbm.at[pl.ds(e0 + ch * ECH, ECH)], esrc_v)
            pltpu.sync_copy(edst_hbm.at[pl.ds(e0 + ch * ECH, ECH)], edst_v)
            lax.fori_loop(0, ECH // L, _group, None)
            pltpu.sync_copy(ex_v, exh_hbm.at[pl.ds(e0 + ch * ECH, ECH)])
            return carry

        lax.fori_loop(0, NCH, _chunk, None)
        plsc.subcore_barrier()
        pltpu.sync_copy(dnloc_v, dn_sp.at[plsc.Indices(idx80_v)], add=True)
        plsc.subcore_barrier()

        @pl.when(s < 10)
        def _():
            pltpu.sync_copy(dn_sp.at[pl.ds(8 * s, 8)], dnh_hbm.at[pl.ds(8 * s, 8)])


def _sc_scores(s_src, s_dst, e_src, e_dst):
    mesh = plsc.VectorSubcoreMesh(
        core_axis_name="c", subcore_axis_name="s", num_cores=NC, num_subcores=NS)
    f = pl.kernel(
        _sc1_body,
        out_type=[
            jax.ShapeDtypeStruct((E,), jnp.float32),
            jax.ShapeDtypeStruct((DNR, 128), jnp.float32),
        ],
        mesh=mesh,
        compiler_params=pltpu.CompilerParams(needs_layout_passes=False),
        scratch_types=[
            pltpu.VMEM_SHARED((DNR, 128), jnp.float32),
            pltpu.VMEM((N,), jnp.float32),
            pltpu.VMEM((N,), jnp.float32),
            pltpu.VMEM((ECH,), jnp.int32),
            pltpu.VMEM((ECH,), jnp.int32),
            pltpu.VMEM((ECH,), jnp.float32),
            pltpu.VMEM((DNR, 128), jnp.float32),
            pltpu.VMEM((DNR,), jnp.int32),
        ],
    )
    zeros2 = jnp.zeros((DNR, 128), jnp.float32)
    return f(s_src, s_dst, e_src, e_dst, zeros2)


def _sc2_body(esrc_hbm, edst_hbm, exh_hbm, h_hbm, z_hbm,
              acch_hbm,
              acc_sp,
              esrc_v, edst_v, exch_v, lt_v,
              r20_v, r21_v,
              gidx0_v, sidxa0_v, sidxb0_v, exm0_v,
              gidx1_v, sidxa1_v, sidxb1_v, exm1_v,
              semg0, sems0, semg1, sems1):
    c = lax.axis_index("c")
    s = lax.axis_index("s")
    lo = (c * HALF).astype(jnp.int32)
    iota = lax.iota(jnp.int32, L)

    r0 = (s * 624).astype(jnp.int32)

    @pl.when(s < 15)
    def _():
        pltpu.sync_copy(z_hbm.at[pl.ds(0, 624)], acc_sp.at[pl.ds(r0, 624)])

    @pl.when(s == 15)
    def _():
        pltpu.sync_copy(z_hbm, acc_sp.at[pl.ds(9360, 640)])

    plsc.subcore_barrier()

    e0 = s * EPT
    slots = (
        (gidx0_v, sidxa0_v, sidxb0_v, exm0_v, r20_v, semg0, sems0),
        (gidx1_v, sidxa1_v, sidxb1_v, exm1_v, r21_v, semg1, sems1),
    )

    def _wait_scatter(slot):
        gidx_v, sidxa_v, sidxb_v, exm_v, r2_v, semg, sems = slot
        pltpu.make_async_copy(
            r2_v.at[pl.ds(0, B)], acc_sp.at[plsc.Indices(sidxa_v)], sems).wait()
        pltpu.make_async_copy(
            r2_v.at[pl.ds(B, B)], acc_sp.at[plsc.Indices(sidxb_v)], sems).wait()

    def _scalar_and_fire(b, slot):
        gidx_v, sidxa_v, sidxb_v, exm_v, r2_v, semg, sems = slot

        @pl.when(b % BPC == 0)
        def _():
            ch = b // BPC
            pltpu.sync_copy(esrc_hbm.at[pl.ds(e0 + ch * ECH, ECH)], esrc_v)
            pltpu.sync_copy(edst_hbm.at[pl.ds(e0 + ch * ECH, ECH)], edst_v)
            pltpu.sync_copy(exh_hbm.at[pl.ds(e0 + ch * ECH, ECH)], exch_v)

        off0 = (b % BPC) * B
        for q in range(B // L):
            off = off0 + q * L
            src16 = esrc_v[pl.ds(off, L)]
            dst16 = edst_v[pl.ds(off, L)]
            ex16 = exch_v[pl.ds(off, L)]
            owned = (dst16 >= lo) & (dst16 < lo + HALF)
            lpos = jnp.where(owned, dst16 - lo, jnp.bitwise_and(dst16, 4095))
            gidx_v[pl.ds(q * L, L)] = src16
            exm_v[pl.ds(q * L, L)] = jnp.where(owned, ex16, 0.0)
            lt_v[...] = lpos
            da = plsc.load_gather(lt_v, [lax.shift_right_logical(iota, 1)])
            va = 2 * da + jnp.bitwise_and(iota, 1)
            db = plsc.load_gather(lt_v, [8 + lax.shift_right_logical(iota, 1)])
            vb = 2 * db + jnp.bitwise_and(iota, 1)
            p = 32 * q
            if p < B:
                sidxa_v[pl.ds(p, L)] = va
            else:
                sidxb_v[pl.ds(p - B, L)] = va
            if p + L < B:
                sidxa_v[pl.ds(p + L, L)] = vb
            else:
                sidxb_v[pl.ds(p + L - B, L)] = vb
        pltpu.async_copy(
            h_hbm.at[plsc.Indices(gidx_v)], r2_v.reshape(B, D), semg)

    def _wait_gather(slot):
        gidx_v, sidxa_v, sidxb_v, exm_v, r2_v, semg, sems = slot
        pltpu.make_async_copy(
            h_hbm.at[plsc.Indices(gidx_v)], r2_v.reshape(B, D), semg).wait()

    def _scale(slot):
        gidx_v, sidxa_v, sidxb_v, exm_v, r2_v, semg, sems = slot

        def _rowgrp(g, carry2):
            for jo in range(L):
                j2 = 2 * (g * L + jo)
                f = plsc.load_gather(exm_v, [jnp.full((L,), jo, jnp.int32) + g * L])
                for k in range(128 // L):
                    r2_v[j2, pl.ds(k * L, L)] = r2_v[j2, pl.ds(k * L, L)] * f
                    r2_v[j2 + 1, pl.ds(k * L, L)] = r2_v[j2 + 1, pl.ds(k * L, L)] * f
            return carry2
        lax.fori_loop(0, B // L, _rowgrp, None)

    def _fire_scatter(slot):
        gidx_v, sidxa_v, sidxb_v, exm_v, r2_v, semg, sems = slot
        pltpu.async_copy(
            r2_v.at[pl.ds(0, B)], acc_sp.at[plsc.Indices(sidxa_v)], sems,
            add=True)
        pltpu.async_copy(
            r2_v.at[pl.ds(B, B)], acc_sp.at[plsc.Indices(sidxb_v)], sems,
            add=True)

    _scalar_and_fire(jnp.int32(0), slots[0])
    _scalar_and_fire(jnp.int32(1), slots[1])

    def _step(i, carry):
        b0 = 2 * i
        _wait_gather(slots[0])
        _scale(slots[0])
        _fire_scatter(slots[0])

        @pl.when(b0 + 2 < NB)
        def _():
            _wait_scatter(slots[0])
            _scalar_and_fire(b0 + 2, slots[0])

        _wait_gather(slots[1])
        _scale(slots[1])
        _fire_scatter(slots[1])

        @pl.when(b0 + 3 < NB)
        def _():
            _wait_scatter(slots[1])
            _scalar_and_fire(b0 + 3, slots[1])
        return carry

    lax.fori_loop(0, NB // 2, _step, None)
    _wait_gather(slots[0])
    _scale(slots[0])
    _fire_scatter(slots[0])
    _wait_scatter(slots[1])
    _wait_scatter(slots[0])
    plsc.subcore_barrier()

    @pl.when(s < 15)
    def _():
        pltpu.sync_copy(acc_sp.at[pl.ds(r0, 624)],
                        acch_hbm.at[pl.ds(c * 2 * HALF + r0, 624)])

    @pl.when(s == 15)
    def _():
        pltpu.sync_copy(acc_sp.at[pl.ds(9360, 640)],
                        acch_hbm.at[pl.ds(c * 2 * HALF + 9360, 640)])


def _sc_aggregate(e_src, e_dst, exh, H):
    mesh = plsc.VectorSubcoreMesh(
        core_axis_name="c", subcore_axis_name="s", num_cores=NC, num_subcores=NS)
    f = pl.kernel(
        _sc2_body,
        out_type=[
            jax.ShapeDtypeStruct((2 * N, 128), jnp.float32),
        ],
        mesh=mesh,
        compiler_params=pltpu.CompilerParams(needs_layout_passes=False),
        scratch_types=[
            pltpu.VMEM_SHARED((2 * HALF, 128), jnp.float32),
            pltpu.VMEM((ECH,), jnp.int32),
            pltpu.VMEM((ECH,), jnp.int32),
            pltpu.VMEM((ECH,), jnp.float32),
            pltpu.VMEM((L,), jnp.int32),
            pltpu.VMEM((2 * B, 128), jnp.float32),
            pltpu.VMEM((2 * B, 128), jnp.float32),
            pltpu.VMEM((B,), jnp.int32),
            pltpu.VMEM((B,), jnp.int32),
            pltpu.VMEM((B,), jnp.int32),
            pltpu.VMEM((B,), jnp.float32),
            pltpu.VMEM((B,), jnp.int32),
            pltpu.VMEM((B,), jnp.int32),
            pltpu.VMEM((B,), jnp.int32),
            pltpu.VMEM((B,), jnp.float32),
            pltpu.SemaphoreType.DMA,
            pltpu.SemaphoreType.DMA,
            pltpu.SemaphoreType.DMA,
            pltpu.SemaphoreType.DMA,
        ],
    )
    zeros = jnp.zeros((640, 128), jnp.float32)
    out = f(e_src, e_dst, exh, H, zeros)
    return out[0] if isinstance(out, (list, tuple)) else out


def _final_body(acc_ref, dn_ref, out_ref):
    inv = 1.0 / (dn_ref[...] + 1e-16)
    y = acc_ref[...] * inv
    out_ref[...] = jnp.where(y > 0, y, jnp.exp(y) - 1.0)


def _final(acc, denom):
    return pl.pallas_call(
        _final_body,
        grid=(N // _BN,),
        in_specs=[
            pl.BlockSpec((_BN, D), lambda i: (i, 0)),
            pl.BlockSpec((_BN, 1), lambda i: (i, 0)),
        ],
        out_specs=pl.BlockSpec((_BN, D), lambda i: (i, 0)),
        out_shape=jax.ShapeDtypeStruct((N, D), jnp.float32),
    )(acc, denom)


def kernel(X, edge_index, W_theta, b_theta, a_src, a_dst):
    H, s_src, s_dst = _compute_h(X, W_theta, b_theta, a_src, a_dst)
    e_src = edge_index[0]
    e_dst = edge_index[1]
    exh, dnh = _sc_scores(s_src[:, 0], s_dst[:, 0], e_src, e_dst)
    acch = _sc_aggregate(e_src, e_dst, exh, H)
    acc = acch.reshape(N, D)
    denom = dnh.reshape(DNR * 128)[:N].reshape(N, 1)
    return _final(acc, denom)
